# Initial kernel scaffold; baseline (speedup 1.0000x reference)
import jax
import jax.numpy as jnp
from jax.experimental import pallas as pl


def kernel(seq, edge_index, training, W, a1, b1, a2, b2, bias_zero):
    raise NotImplementedError("write your pallas kernel here")


# speedup vs baseline: 20.3908x; 20.3908x over previous
import functools

import jax
import jax.numpy as jnp
from jax import lax
from jax.experimental import pallas as pl
from jax.experimental.pallas import tpu as pltpu
from jax.experimental.pallas import tpu_sc as plsc

N = 10000
E = 320000
D = 128
H = 128

NC = 2
NS = 16
NW = NC * NS
L = 16

NP = 10240
EPT = E // NW
C = 80
NCHUNK = EPT // C

TCB = 2000


def _tc1_body(x_ref, w_ref, a_ref, b_ref, sfts_ref, f12_ref, m_ref):
    i = pl.program_id(0)
    s = jnp.dot(x_ref[...], w_ref[...],
                precision=lax.Precision.HIGHEST,
                preferred_element_type=jnp.float32)
    sfts_ref[...] = s
    f = jnp.dot(s, a_ref[...],
                precision=lax.Precision.HIGHEST,
                preferred_element_type=jnp.float32) + b_ref[...]
    f12_ref[...] = f
    m = jnp.max(f, axis=0, keepdims=True)

    @pl.when(i == 0)
    def _():
        m_ref[...] = m

    @pl.when(i != 0)
    def _():
        m_ref[...] = jnp.maximum(m_ref[...], m)


def _tc1(x, W, A, b2d):
    return pl.pallas_call(
        _tc1_body,
        grid=(N // TCB,),
        in_specs=[
            pl.BlockSpec((TCB, D), lambda i: (i, 0)),
            pl.BlockSpec((D, H), lambda i: (0, 0)),
            pl.BlockSpec((H, 2), lambda i: (0, 0)),
            pl.BlockSpec((1, 2), lambda i: (0, 0)),
        ],
        out_specs=[
            pl.BlockSpec((TCB, H), lambda i: (i, 0)),
            pl.BlockSpec((TCB, 2), lambda i: (i, 0)),
            pl.BlockSpec((1, 2), lambda i: (0, 0)),
        ],
        out_shape=[
            jax.ShapeDtypeStruct((N, H), jnp.float32),
            jax.ShapeDtypeStruct((N, 2), jnp.float32),
            jax.ShapeDtypeStruct((1, 2), jnp.float32),
        ],
    )(x, W, A, b2d)


_MESH = plsc.VectorSubcoreMesh(
    core_axis_name="c", subcore_axis_name="s", num_cores=NC, num_subcores=NS)


def _sc1_body(rows_hbm, cols_hbm, f1_hbm, f2_hbm, m_hbm, sfts_hbm,
              vals0_out, vals1_out, den0_out, den1_out,
              f1_v, f2_v, m_v, rows_v, cols_v, gath_v, ex16_v,
              dent,
              vals_sh, den16_sh, gsem):
    core = lax.axis_index("c")
    sid = lax.axis_index("s")
    zeros16f = jnp.zeros((L,), jnp.float32)
    zeros16i = jnp.zeros((L,), jnp.int32)
    ones16i = jnp.ones((L,), jnp.int32)
    iota16 = lax.iota(jnp.int32, L)

    pltpu.sync_copy(f1_hbm, f1_v)
    pltpu.sync_copy(f2_hbm, f2_v)
    pltpu.sync_copy(m_hbm, m_v)
    mrow = m_v[:]
    c_shift = jnp.maximum(mrow[0] + mrow[1], 0.0)

    def _z1(r, _):
        for j in range(D // L):
            gath_v[r, pl.ds(j * L, L)] = zeros16f
        ex16_v[r, :] = zeros16f
        return 0
    lax.fori_loop(0, C, _z1, 0)

    vrows = NP // NS
    drows = NP // NS
    for k in range(drows // C):
        pltpu.sync_copy(gath_v, vals_sh.at[pl.ds(sid * vrows + k * C, C), :])
        pltpu.sync_copy(ex16_v, den16_sh.at[pl.ds(sid * drows + k * C, C), :])
    plsc.subcore_barrier()

    tile_base = (core * NS + sid) * EPT

    def _chunk(k, _):
        base = tile_base + k * C
        pltpu.sync_copy(rows_hbm.at[pl.ds(base, C)], rows_v)
        pltpu.sync_copy(cols_hbm.at[pl.ds(base, C)], cols_v)
        pltpu.async_copy(sfts_hbm.at[cols_v], gath_v, gsem).wait()

        for g in range(C // L):
            rvec = rows_v[pl.ds(g * L, L)]
            cvec = cols_v[pl.ds(g * L, L)]
            f1g = plsc.load_gather(f1_v, [rvec])
            f2g = plsc.load_gather(f2_v, [cvec])
            e = f1g + f2g
            e = jnp.where(e >= 0.0, e, 0.2 * e) - c_shift
            ex = jnp.exp(e)
            plsc.store_scatter(ex16_v, [iota16 + g * L, zeros16i], ex)

        def _scale(ei, _):
            exrow = ex16_v[ei, :]
            wv = jnp.full((L,), exrow[0], jnp.float32)
            for j in range(D // L):
                gath_v[ei, pl.ds(j * L, L)] = gath_v[ei, pl.ds(j * L, L)] * wv
            return 0
        lax.fori_loop(0, C, _scale, 0)

        pltpu.sync_copy(gath_v, vals_sh.at[rows_v], add=True)
        pltpu.sync_copy(ex16_v, den16_sh.at[rows_v], add=True)
        return 0

    lax.fori_loop(0, NCHUNK, _chunk, 0)
    plsc.subcore_barrier()

    for k in range(drows // C):
        pltpu.sync_copy(den16_sh.at[pl.ds(sid * drows + k * C, C), :], ex16_v)
        for g in range(C // L):
            dv = plsc.load_gather(ex16_v, [iota16 + g * L, zeros16i])
            dent[pl.ds(k * C + g * L, L)] = dv

    @pl.when(core == 0)
    def _():
        pltpu.sync_copy(vals_sh.at[pl.ds(sid * vrows, vrows), :],
                        vals0_out.at[pl.ds(sid * vrows, vrows), :])
        pltpu.sync_copy(dent, den0_out.at[pl.ds(sid * drows, drows)])

    @pl.when(core == 1)
    def _():
        pltpu.sync_copy(vals_sh.at[pl.ds(sid * vrows, vrows), :],
                        vals1_out.at[pl.ds(sid * vrows, vrows), :])
        pltpu.sync_copy(dent, den1_out.at[pl.ds(sid * drows, drows)])


_sc1 = functools.partial(
    pl.kernel, _sc1_body,
    compiler_params=pltpu.CompilerParams(needs_layout_passes=False, use_tc_tiling_on_sc=False),
    out_type=(
        jax.ShapeDtypeStruct((NP, H), jnp.float32),
        jax.ShapeDtypeStruct((NP, H), jnp.float32),
        jax.ShapeDtypeStruct((NP,), jnp.float32),
        jax.ShapeDtypeStruct((NP,), jnp.float32),
    ),
    mesh=_MESH,
    scratch_types=(
        pltpu.VMEM((N,), jnp.float32),
        pltpu.VMEM((N,), jnp.float32),
        pltpu.VMEM((L,), jnp.float32),
        pltpu.VMEM((C,), jnp.int32),
        pltpu.VMEM((C,), jnp.int32),
        pltpu.VMEM((C, D), jnp.float32),
        pltpu.VMEM((C, L), jnp.float32),
        pltpu.VMEM((640,), jnp.float32),
        pltpu.VMEM_SHARED((NP, H), jnp.float32),
        pltpu.VMEM_SHARED((NP, L), jnp.float32),
        pltpu.SemaphoreType.DMA,
    ),
)()


RB = 200
NRC = N // RB


def _sc2_body(vals0_hbm, vals1_hbm, den0_hbm, den1_hbm, bias_hbm, out_hbm,
              v0_v, v1_v, d0_v, d1_v, recip_v, bias_v):
    core = lax.axis_index("c")
    sid = lax.axis_index("s")
    wid = core * NS + sid
    pltpu.sync_copy(bias_hbm, bias_v)

    def _do(cid):
        r0 = cid * RB
        pltpu.sync_copy(vals0_hbm.at[pl.ds(r0, RB), :], v0_v)
        pltpu.sync_copy(vals1_hbm.at[pl.ds(r0, RB), :], v1_v)
        pltpu.sync_copy(den0_hbm.at[pl.ds(r0, RB)], d0_v.at[pl.ds(0, RB)])
        pltpu.sync_copy(den1_hbm.at[pl.ds(r0, RB)], d1_v.at[pl.ds(0, RB)])
        for g in range(208 // L):
            sl = pl.ds(g * L, L)
            dv = d0_v[sl] + d1_v[sl]
            recip_v[sl] = jnp.where(dv != 0.0, 1.0 / dv, 0.0)

        def _row(r, _):
            rv = recip_v[pl.ds(r, L)]
            wv = jnp.full((L,), rv[0], jnp.float32)
            for j in range(H // L):
                sl = pl.ds(j * L, L)
                o = (v0_v[r, sl] + v1_v[r, sl]) * wv + bias_v[sl]
                o = jnp.where(o > 0.0, o, jnp.exp(jnp.minimum(o, 0.0)) - 1.0)
                v0_v[r, sl] = o
            return 0
        lax.fori_loop(0, RB, _row, 0)
        pltpu.sync_copy(v0_v, out_hbm.at[0, pl.ds(r0, RB), :])

    for rep in range(2):
        cid = wid + NW * rep

        @pl.when(cid < NRC)
        def _():
            _do(cid)


_sc2 = functools.partial(
    pl.kernel, _sc2_body,
    compiler_params=pltpu.CompilerParams(needs_layout_passes=False, use_tc_tiling_on_sc=False),
    out_type=jax.ShapeDtypeStruct((1, N, H), jnp.float32),
    mesh=_MESH,
    scratch_types=(
        pltpu.VMEM((RB, H), jnp.float32),
        pltpu.VMEM((RB, H), jnp.float32),
        pltpu.VMEM((208,), jnp.float32),
        pltpu.VMEM((208,), jnp.float32),
        pltpu.VMEM((224,), jnp.float32),
        pltpu.VMEM((H,), jnp.float32),
    ),
)()


def kernel(seq, edge_index, training, W, a1, b1, a2, b2, bias_zero):
    x = seq[0]
    rows = edge_index[0]
    cols = edge_index[1]
    A = jnp.concatenate([a1, a2], axis=1)
    b2d = jnp.concatenate([b1, b2]).reshape(1, 2)
    sfts, f12, m12 = _tc1(x, W, A, b2d)
    f1 = f12[:, 0]
    f2 = f12[:, 1]
    mpad = jnp.pad(m12.reshape(2), (0, L - 2))
    vals0, vals1, den0, den1 = _sc1(rows, cols, f1, f2, mpad, sfts)
    return _sc2(vals0, vals1, den0, den1, bias_zero)


# speedup vs baseline: 37.4997x; 1.8390x over previous
import functools

import jax
import jax.numpy as jnp
from jax import lax
from jax.experimental import pallas as pl
from jax.experimental.pallas import tpu as pltpu
from jax.experimental.pallas import tpu_sc as plsc

N = 10000
E = 320000
D = 128
H = 128

NC = 2
NS = 16
NW = NC * NS
L = 16

NP = 10240
EPT = E // NW
C = 80
NCHUNK = EPT // C
CA = 400
NCA = EPT // CA

TCB = 2000


def _tc1_body(x_ref, w_ref, a_ref, b_ref, sfts_ref, f12_ref, m_ref):
    i = pl.program_id(0)
    s = jnp.dot(x_ref[...], w_ref[...],
                precision=lax.Precision.HIGHEST,
                preferred_element_type=jnp.float32)
    sfts_ref[...] = s
    f = jnp.dot(s, a_ref[...],
                precision=lax.Precision.HIGHEST,
                preferred_element_type=jnp.float32) + b_ref[...]
    f12_ref[...] = f
    m = jnp.max(f, axis=0, keepdims=True)

    @pl.when(i == 0)
    def _():
        m_ref[...] = m

    @pl.when(i != 0)
    def _():
        m_ref[...] = jnp.maximum(m_ref[...], m)


def _tc1(x, W, A, b2d):
    return pl.pallas_call(
        _tc1_body,
        grid=(N // TCB,),
        in_specs=[
            pl.BlockSpec((TCB, D), lambda i: (i, 0)),
            pl.BlockSpec((D, H), lambda i: (0, 0)),
            pl.BlockSpec((H, 2), lambda i: (0, 0)),
            pl.BlockSpec((1, 2), lambda i: (0, 0)),
        ],
        out_specs=[
            pl.BlockSpec((TCB, H), lambda i: (i, 0)),
            pl.BlockSpec((TCB, 2), lambda i: (i, 0)),
            pl.BlockSpec((1, 2), lambda i: (0, 0)),
        ],
        out_shape=[
            jax.ShapeDtypeStruct((N, H), jnp.float32),
            jax.ShapeDtypeStruct((N, 2), jnp.float32),
            jax.ShapeDtypeStruct((1, 2), jnp.float32),
        ],
    )(x, W, A, b2d)


_MESH = plsc.VectorSubcoreMesh(
    core_axis_name="c", subcore_axis_name="s", num_cores=NC, num_subcores=NS)

_SC_PARAMS = pltpu.CompilerParams(
    needs_layout_passes=False, use_tc_tiling_on_sc=False)


def _sc1a_body(rows_hbm, cols_hbm, f1_hbm, f2_hbm, m_hbm, ex_out,
               f1_v, f2_v, m_v, r0, r1, c0, c1, e0, e1, isem, osem):
    core = lax.axis_index("c")
    sid = lax.axis_index("s")
    abase = (core * NS + sid) * EPT
    RV = [r0, r1]
    CV = [c0, c1]
    EX = [e0, e1]

    pltpu.sync_copy(f1_hbm, f1_v)
    pltpu.sync_copy(f2_hbm, f2_v)
    pltpu.sync_copy(m_hbm, m_v)
    mrow = m_v[:]
    c_shift = jnp.maximum(mrow[0] + mrow[1], 0.0)

    def issue_idx(t):
        sl = pl.ds(abase + t * CA, CA)
        pltpu.async_copy(rows_hbm.at[sl], RV[t % 2], isem)
        pltpu.async_copy(cols_hbm.at[sl], CV[t % 2], isem)

    def wait_idx(t):
        sl = pl.ds(abase + t * CA, CA)
        pltpu.make_async_copy(rows_hbm.at[sl], RV[t % 2], isem).wait()
        pltpu.make_async_copy(cols_hbm.at[sl], CV[t % 2], isem).wait()

    issue_idx(0)
    for t in range(NCA):
        X = t % 2
        wait_idx(t)
        if t + 1 < NCA:
            issue_idx(t + 1)
        if t >= 2:
            pltpu.make_async_copy(
                EX[X], ex_out.at[pl.ds(abase + (t - 2) * CA, CA)], osem).wait()

        def _grp(g, _):
            sl = pl.ds(g * L, L)
            rvec = RV[X][sl]
            cvec = CV[X][sl]
            e = plsc.load_gather(f1_v, [rvec]) + plsc.load_gather(f2_v, [cvec])
            e = jnp.where(e >= 0.0, e, 0.2 * e) - c_shift
            EX[X][sl] = jnp.exp(e)
            return 0
        lax.fori_loop(0, CA // L, _grp, 0)
        pltpu.async_copy(EX[X], ex_out.at[pl.ds(abase + t * CA, CA)], osem)

    for t in (NCA - 2, NCA - 1):
        pltpu.make_async_copy(
            EX[t % 2], ex_out.at[pl.ds(abase + t * CA, CA)], osem).wait()


_sc1a = functools.partial(
    pl.kernel, _sc1a_body,
    compiler_params=_SC_PARAMS,
    out_type=jax.ShapeDtypeStruct((E,), jnp.float32),
    mesh=_MESH,
    scratch_types=(
        pltpu.VMEM((N,), jnp.float32),
        pltpu.VMEM((N,), jnp.float32),
        pltpu.VMEM((L,), jnp.float32),
        pltpu.VMEM((CA,), jnp.int32),
        pltpu.VMEM((CA,), jnp.int32),
        pltpu.VMEM((CA,), jnp.int32),
        pltpu.VMEM((CA,), jnp.int32),
        pltpu.VMEM((CA,), jnp.float32),
        pltpu.VMEM((CA,), jnp.float32),
        pltpu.SemaphoreType.DMA,
        pltpu.SemaphoreType.DMA,
    ),
)()


def _sc1b_body(rows_hbm, cols_hbm, ex_hbm, sfts_hbm,
               vals0_out, vals1_out, den0_out, den1_out,
               rv0, rv1, rv2, cv0, cv1, cv2, eb0, eb1, eb2,
               rs0, rs1, rs2, g0, g1, g2, x0, x1, x2, dent,
               vals_sh, den16_sh, gsem, isem, ssem0, ssem1, ssem2):
    core = lax.axis_index("c")
    sid = lax.axis_index("s")
    tbase = (core * NS + sid) * EPT
    RV = [rv0, rv1, rv2]
    CV = [cv0, cv1, cv2]
    EB = [eb0, eb1, eb2]
    RS = [rs0, rs1, rs2]
    G = [g0, g1, g2]
    X16 = [x0, x1, x2]
    SS = [ssem0, ssem1, ssem2]
    zeros16f = jnp.zeros((L,), jnp.float32)
    zeros16i = jnp.zeros((L,), jnp.int32)
    iota16 = lax.iota(jnp.int32, L)

    def _z1(r, _):
        for j in range(D // L):
            g0[r, pl.ds(j * L, L)] = zeros16f
        x0[r, :] = zeros16f
        return 0
    lax.fori_loop(0, C, _z1, 0)
    rows_per_tile = NP // NS
    for k in range(rows_per_tile // C):
        pltpu.sync_copy(g0, vals_sh.at[pl.ds(sid * rows_per_tile + k * C, C), :])
        pltpu.sync_copy(x0, den16_sh.at[pl.ds(sid * rows_per_tile + k * C, C), :])
    plsc.subcore_barrier()

    def issue_idx(k, P):
        sl = pl.ds(tbase + k * C, C)
        pltpu.async_copy(rows_hbm.at[sl], RV[P], isem)
        pltpu.async_copy(cols_hbm.at[sl], CV[P], isem)
        pltpu.async_copy(ex_hbm.at[sl], EB[P], isem)

    def wait_idx(k, P):
        sl = pl.ds(tbase + k * C, C)
        pltpu.make_async_copy(rows_hbm.at[sl], RV[P], isem).wait()
        pltpu.make_async_copy(cols_hbm.at[sl], CV[P], isem).wait()
        pltpu.make_async_copy(ex_hbm.at[sl], EB[P], isem).wait()

    def wait_scatter(R):
        pltpu.make_async_copy(G[R], vals_sh.at[RS[R]], SS[R]).wait()
        pltpu.make_async_copy(X16[R], den16_sh.at[RS[R]], SS[R]).wait()

    issue_idx(0, 0)
    issue_idx(1, 1)
    wait_idx(0, 0)
    pltpu.async_copy(sfts_hbm.at[CV[0]], G[0], gsem)

    def _body(k, P, Q, R):
        pltpu.make_async_copy(sfts_hbm.at[CV[P]], G[P], gsem).wait()

        @pl.when(k + 1 < NCHUNK)
        def _():
            wait_idx(k + 1, Q)

        @pl.when(k + 2 < NCHUNK)
        def _():
            issue_idx(k + 2, R)

        @pl.when(k >= 1)
        def _():
            wait_scatter(R)

        @pl.when(k + 1 < NCHUNK)
        def _():
            pltpu.async_copy(sfts_hbm.at[CV[Q]], G[Q], gsem)

        for g in range(C // L):
            sl = pl.ds(g * L, L)
            RS[P][sl] = RV[P][sl]

        def _grp(g, _):
            exvec = EB[P][pl.ds(g * L, L)]
            plsc.store_scatter(X16[P], [iota16 + g * L, zeros16i], exvec)
            for lane in range(L):
                wv = jnp.full((L,), exvec[lane], jnp.float32)
                ei = g * L + lane
                for j in range(D // L):
                    sl = pl.ds(j * L, L)
                    G[P][ei, sl] = G[P][ei, sl] * wv
            return 0
        lax.fori_loop(0, C // L, _grp, 0)

        pltpu.async_copy(G[P], vals_sh.at[RS[P]], SS[P], add=True)
        pltpu.async_copy(X16[P], den16_sh.at[RS[P]], SS[P], add=True)

    def _full(k, _):
        for ph in range(3):
            @pl.when(k % 3 == ph)
            def _():
                _body(k, ph, (ph + 1) % 3, (ph + 2) % 3)
        return 0
    lax.fori_loop(0, NCHUNK, _full, 0)

    wait_scatter((NCHUNK - 1) % 3)
    plsc.subcore_barrier()

    drows = NP // NS
    for kk in range(drows // C):
        pltpu.sync_copy(den16_sh.at[pl.ds(sid * drows + kk * C, C), :], x0)
        for g in range(C // L):
            dv = plsc.load_gather(x0, [iota16 + g * L, zeros16i])
            dent[pl.ds(kk * C + g * L, L)] = dv

    @pl.when(core == 0)
    def _():
        pltpu.sync_copy(vals_sh.at[pl.ds(sid * drows, drows), :],
                        vals0_out.at[pl.ds(sid * drows, drows), :])
        pltpu.sync_copy(dent, den0_out.at[pl.ds(sid * drows, drows)])

    @pl.when(core == 1)
    def _():
        pltpu.sync_copy(vals_sh.at[pl.ds(sid * drows, drows), :],
                        vals1_out.at[pl.ds(sid * drows, drows), :])
        pltpu.sync_copy(dent, den1_out.at[pl.ds(sid * drows, drows)])


_sc1b = functools.partial(
    pl.kernel, _sc1b_body,
    compiler_params=_SC_PARAMS,
    out_type=(
        jax.ShapeDtypeStruct((NP, H), jnp.float32),
        jax.ShapeDtypeStruct((NP, H), jnp.float32),
        jax.ShapeDtypeStruct((NP,), jnp.float32),
        jax.ShapeDtypeStruct((NP,), jnp.float32),
    ),
    mesh=_MESH,
    scratch_types=(
        pltpu.VMEM((C,), jnp.int32),
        pltpu.VMEM((C,), jnp.int32),
        pltpu.VMEM((C,), jnp.int32),
        pltpu.VMEM((C,), jnp.int32),
        pltpu.VMEM((C,), jnp.int32),
        pltpu.VMEM((C,), jnp.int32),
        pltpu.VMEM((C,), jnp.float32),
        pltpu.VMEM((C,), jnp.float32),
        pltpu.VMEM((C,), jnp.float32),
        pltpu.VMEM((C,), jnp.int32),
        pltpu.VMEM((C,), jnp.int32),
        pltpu.VMEM((C,), jnp.int32),
        pltpu.VMEM((C, D), jnp.float32),
        pltpu.VMEM((C, D), jnp.float32),
        pltpu.VMEM((C, D), jnp.float32),
        pltpu.VMEM((C, L), jnp.float32),
        pltpu.VMEM((C, L), jnp.float32),
        pltpu.VMEM((C, L), jnp.float32),
        pltpu.VMEM((NP // NS,), jnp.float32),
        pltpu.VMEM_SHARED((NP, H), jnp.float32),
        pltpu.VMEM_SHARED((NP, L), jnp.float32),
        pltpu.SemaphoreType.DMA,
        pltpu.SemaphoreType.DMA,
        pltpu.SemaphoreType.DMA,
        pltpu.SemaphoreType.DMA,
        pltpu.SemaphoreType.DMA,
    ),
)()


RB = 200
NRC = N // RB


def _sc2_body(vals0_hbm, vals1_hbm, den0_hbm, den1_hbm, bias_hbm, out_hbm,
              v0_v, v1_v, d0_v, d1_v, recip_v, bias_v):
    core = lax.axis_index("c")
    sid = lax.axis_index("s")
    wid = core * NS + sid
    pltpu.sync_copy(bias_hbm, bias_v)

    def _do(cid):
        r0 = cid * RB
        pltpu.sync_copy(vals0_hbm.at[pl.ds(r0, RB), :], v0_v)
        pltpu.sync_copy(vals1_hbm.at[pl.ds(r0, RB), :], v1_v)
        pltpu.sync_copy(den0_hbm.at[pl.ds(r0, RB)], d0_v.at[pl.ds(0, RB)])
        pltpu.sync_copy(den1_hbm.at[pl.ds(r0, RB)], d1_v.at[pl.ds(0, RB)])
        for g in range(208 // L):
            sl = pl.ds(g * L, L)
            dv = d0_v[sl] + d1_v[sl]
            recip_v[sl] = jnp.where(dv != 0.0, 1.0 / dv, 0.0)

        def _row(r, _):
            rv = recip_v[pl.ds(r, L)]
            wv = jnp.full((L,), rv[0], jnp.float32)
            for j in range(H // L):
                sl = pl.ds(j * L, L)
                o = (v0_v[r, sl] + v1_v[r, sl]) * wv + bias_v[sl]
                o = jnp.where(o > 0.0, o, jnp.exp(jnp.minimum(o, 0.0)) - 1.0)
                v0_v[r, sl] = o
            return 0
        lax.fori_loop(0, RB, _row, 0)
        pltpu.sync_copy(v0_v, out_hbm.at[0, pl.ds(r0, RB), :])

    for rep in range(2):
        cid = wid + NW * rep

        @pl.when(cid < NRC)
        def _():
            _do(cid)


_sc2 = functools.partial(
    pl.kernel, _sc2_body,
    compiler_params=_SC_PARAMS,
    out_type=jax.ShapeDtypeStruct((1, N, H), jnp.float32),
    mesh=_MESH,
    scratch_types=(
        pltpu.VMEM((RB, H), jnp.float32),
        pltpu.VMEM((RB, H), jnp.float32),
        pltpu.VMEM((208,), jnp.float32),
        pltpu.VMEM((208,), jnp.float32),
        pltpu.VMEM((224,), jnp.float32),
        pltpu.VMEM((H,), jnp.float32),
    ),
)()


def kernel(seq, edge_index, training, W, a1, b1, a2, b2, bias_zero):
    x = seq[0]
    rows = edge_index[0]
    cols = edge_index[1]
    A = jnp.concatenate([a1, a2], axis=1)
    b2d = jnp.concatenate([b1, b2]).reshape(1, 2)
    sfts, f12, m12 = _tc1(x, W, A, b2d)
    f1 = f12[:, 0]
    f2 = f12[:, 1]
    mpad = jnp.pad(m12.reshape(2), (0, L - 2))
    ex = _sc1a(rows, cols, f1, f2, mpad)
    vals0, vals1, den0, den1 = _sc1b(rows, cols, ex, sfts)
    return _sc2(vals0, vals1, den0, den1, bias_zero)


# speedup vs baseline: 41.1878x; 1.0983x over previous
import functools

import jax
import jax.numpy as jnp
from jax import lax
from jax.experimental import pallas as pl
from jax.experimental.pallas import tpu as pltpu
from jax.experimental.pallas import tpu_sc as plsc

N = 10000
E = 320000
D = 128
H = 128

NC = 2
NS = 16
NW = NC * NS
L = 16

NP = 10240
EPT = E // NW
C = 80
NCHUNK = EPT // C
CA = 400
NCA = EPT // CA

TCB = 2000


def _tc1_body(x_ref, w_ref, a_ref, b_ref, sfts_ref, f12_ref, m_ref):
    i = pl.program_id(0)
    s = jnp.dot(x_ref[...], w_ref[...],
                precision=lax.Precision.HIGHEST,
                preferred_element_type=jnp.float32)
    sfts_ref[...] = s
    f = jnp.dot(s, a_ref[...],
                precision=lax.Precision.HIGHEST,
                preferred_element_type=jnp.float32) + b_ref[...]
    f12_ref[...] = f
    m = jnp.max(f, axis=0, keepdims=True)

    @pl.when(i == 0)
    def _():
        m_ref[...] = m

    @pl.when(i != 0)
    def _():
        m_ref[...] = jnp.maximum(m_ref[...], m)


def _tc1(x, W, A, b2d):
    return pl.pallas_call(
        _tc1_body,
        grid=(N // TCB,),
        in_specs=[
            pl.BlockSpec((TCB, D), lambda i: (i, 0)),
            pl.BlockSpec((D, H), lambda i: (0, 0)),
            pl.BlockSpec((H, 2), lambda i: (0, 0)),
            pl.BlockSpec((1, 2), lambda i: (0, 0)),
        ],
        out_specs=[
            pl.BlockSpec((TCB, H), lambda i: (i, 0)),
            pl.BlockSpec((TCB, 2), lambda i: (i, 0)),
            pl.BlockSpec((1, 2), lambda i: (0, 0)),
        ],
        out_shape=[
            jax.ShapeDtypeStruct((N, H), jnp.float32),
            jax.ShapeDtypeStruct((N, 2), jnp.float32),
            jax.ShapeDtypeStruct((1, 2), jnp.float32),
        ],
    )(x, W, A, b2d)


_MESH = plsc.VectorSubcoreMesh(
    core_axis_name="c", subcore_axis_name="s", num_cores=NC, num_subcores=NS)

_SC_PARAMS = pltpu.CompilerParams(
    needs_layout_passes=False, use_tc_tiling_on_sc=False)


def _sc1b_body(rows_hbm, cols_hbm, f1_hbm, f2_hbm, m_hbm, sfts_hbm,
               vals0_out, vals1_out, den0_out, den1_out,
               rv0, rv1, rv2, cv0, cv1, cv2,
               fa0, fa1, fa2, fb0, fb1, fb2, m_v,
               rs0, rs1, rs2, g0, g1, g2, x0, x1, x2, dent,
               vals_sh, den16_sh, gsem, isem, fsem, ssem0, ssem1, ssem2):
    core = lax.axis_index("c")
    sid = lax.axis_index("s")
    tbase = (core * NS + sid) * EPT
    RV = [rv0, rv1, rv2]
    CV = [cv0, cv1, cv2]
    FA = [fa0, fa1, fa2]
    FB = [fb0, fb1, fb2]
    RS = [rs0, rs1, rs2]
    G = [g0, g1, g2]
    X16 = [x0, x1, x2]
    SS = [ssem0, ssem1, ssem2]
    zeros16f = jnp.zeros((L,), jnp.float32)
    zeros16i = jnp.zeros((L,), jnp.int32)
    iota16 = lax.iota(jnp.int32, L)

    def _z1(r, _):
        for j in range(D // L):
            g0[r, pl.ds(j * L, L)] = zeros16f
        x0[r, :] = zeros16f
        return 0
    lax.fori_loop(0, C, _z1, 0)
    rows_per_tile = NP // NS
    for k in range(rows_per_tile // C):
        pltpu.sync_copy(g0, vals_sh.at[pl.ds(sid * rows_per_tile + k * C, C), :])
        pltpu.sync_copy(x0, den16_sh.at[pl.ds(sid * rows_per_tile + k * C, C), :])
    plsc.subcore_barrier()

    def issue_idx(k, P):
        sl = pl.ds(tbase + k * C, C)
        pltpu.async_copy(rows_hbm.at[sl], RV[P], isem)
        pltpu.async_copy(cols_hbm.at[sl], CV[P], isem)

    def wait_idx(k, P):
        sl = pl.ds(tbase + k * C, C)
        pltpu.make_async_copy(rows_hbm.at[sl], RV[P], isem).wait()
        pltpu.make_async_copy(cols_hbm.at[sl], CV[P], isem).wait()

    def issue_fg(P):
        pltpu.async_copy(f1_hbm.at[RV[P]], FA[P], fsem)
        pltpu.async_copy(f2_hbm.at[CV[P]], FB[P], fsem)

    def wait_fg(P):
        pltpu.make_async_copy(f1_hbm.at[RV[P]], FA[P], fsem).wait()
        pltpu.make_async_copy(f2_hbm.at[CV[P]], FB[P], fsem).wait()

    def wait_scatter(R):
        pltpu.make_async_copy(G[R], vals_sh.at[RS[R]], SS[R]).wait()
        pltpu.make_async_copy(X16[R], den16_sh.at[RS[R]], SS[R]).wait()

    pltpu.sync_copy(m_hbm, m_v)
    mrow = m_v[:]
    c_shift = jnp.maximum(mrow[0] + mrow[1], 0.0)

    issue_idx(0, 0)
    issue_idx(1, 1)
    wait_idx(0, 0)
    issue_fg(0)
    pltpu.async_copy(sfts_hbm.at[CV[0]], G[0], gsem)

    def _body(k, P, Q, R):
        pltpu.make_async_copy(sfts_hbm.at[CV[P]], G[P], gsem).wait()
        wait_fg(P)

        @pl.when(k + 1 < NCHUNK)
        def _():
            wait_idx(k + 1, Q)
            issue_fg(Q)

        @pl.when(k + 2 < NCHUNK)
        def _():
            issue_idx(k + 2, R)

        @pl.when(k >= 1)
        def _():
            wait_scatter(R)

        @pl.when(k + 1 < NCHUNK)
        def _():
            pltpu.async_copy(sfts_hbm.at[CV[Q]], G[Q], gsem)

        for g in range(C // L):
            sl = pl.ds(g * L, L)
            RS[P][sl] = RV[P][sl]

        def _grp(g, _):
            sl16 = pl.ds(g * L, L)
            e = FA[P][sl16] + FB[P][sl16]
            e = jnp.where(e >= 0.0, e, 0.2 * e) - c_shift
            exvec = jnp.exp(e)
            plsc.store_scatter(X16[P], [iota16 + g * L, zeros16i], exvec)
            for lane in range(L):
                wv = jnp.full((L,), exvec[lane], jnp.float32)
                ei = g * L + lane
                for j in range(D // L):
                    sl = pl.ds(j * L, L)
                    G[P][ei, sl] = G[P][ei, sl] * wv
            return 0
        lax.fori_loop(0, C // L, _grp, 0)

        pltpu.async_copy(G[P], vals_sh.at[RS[P]], SS[P], add=True)
        pltpu.async_copy(X16[P], den16_sh.at[RS[P]], SS[P], add=True)

    def _full(k, _):
        for ph in range(3):
            @pl.when(k % 3 == ph)
            def _():
                _body(k, ph, (ph + 1) % 3, (ph + 2) % 3)
        return 0
    lax.fori_loop(0, NCHUNK, _full, 0)

    wait_scatter((NCHUNK - 1) % 3)
    plsc.subcore_barrier()

    drows = NP // NS
    for kk in range(drows // C):
        pltpu.sync_copy(den16_sh.at[pl.ds(sid * drows + kk * C, C), :], x0)
        for g in range(C // L):
            dv = plsc.load_gather(x0, [iota16 + g * L, zeros16i])
            dent[pl.ds(kk * C + g * L, L)] = dv

    @pl.when(core == 0)
    def _():
        pltpu.sync_copy(vals_sh.at[pl.ds(sid * drows, drows), :],
                        vals0_out.at[pl.ds(sid * drows, drows), :])
        pltpu.sync_copy(dent, den0_out.at[pl.ds(sid * drows, drows)])

    @pl.when(core == 1)
    def _():
        pltpu.sync_copy(vals_sh.at[pl.ds(sid * drows, drows), :],
                        vals1_out.at[pl.ds(sid * drows, drows), :])
        pltpu.sync_copy(dent, den1_out.at[pl.ds(sid * drows, drows)])


_sc1b = functools.partial(
    pl.kernel, _sc1b_body,
    compiler_params=_SC_PARAMS,
    out_type=(
        jax.ShapeDtypeStruct((NP, H), jnp.float32),
        jax.ShapeDtypeStruct((NP, H), jnp.float32),
        jax.ShapeDtypeStruct((NP,), jnp.float32),
        jax.ShapeDtypeStruct((NP,), jnp.float32),
    ),
    mesh=_MESH,
    scratch_types=(
        pltpu.VMEM((C,), jnp.int32),
        pltpu.VMEM((C,), jnp.int32),
        pltpu.VMEM((C,), jnp.int32),
        pltpu.VMEM((C,), jnp.int32),
        pltpu.VMEM((C,), jnp.int32),
        pltpu.VMEM((C,), jnp.int32),
        pltpu.VMEM((C,), jnp.float32),
        pltpu.VMEM((C,), jnp.float32),
        pltpu.VMEM((C,), jnp.float32),
        pltpu.VMEM((C,), jnp.float32),
        pltpu.VMEM((C,), jnp.float32),
        pltpu.VMEM((C,), jnp.float32),
        pltpu.VMEM((L,), jnp.float32),
        pltpu.VMEM((C,), jnp.int32),
        pltpu.VMEM((C,), jnp.int32),
        pltpu.VMEM((C,), jnp.int32),
        pltpu.VMEM((C, D), jnp.float32),
        pltpu.VMEM((C, D), jnp.float32),
        pltpu.VMEM((C, D), jnp.float32),
        pltpu.VMEM((C, L), jnp.float32),
        pltpu.VMEM((C, L), jnp.float32),
        pltpu.VMEM((C, L), jnp.float32),
        pltpu.VMEM((NP // NS,), jnp.float32),
        pltpu.VMEM_SHARED((NP, H), jnp.float32),
        pltpu.VMEM_SHARED((NP, L), jnp.float32),
        pltpu.SemaphoreType.DMA,
        pltpu.SemaphoreType.DMA,
        pltpu.SemaphoreType.DMA,
        pltpu.SemaphoreType.DMA,
        pltpu.SemaphoreType.DMA,
        pltpu.SemaphoreType.DMA,
    ),
)()


RB = 200
NRC = N // RB


def _sc2_body(vals0_hbm, vals1_hbm, den0_hbm, den1_hbm, bias_hbm, out_hbm,
              v0a, v1a, v0b, v1b, d0a, d1a, d0b, d1b, recip_v, bias_v,
              insem, outsem):
    core = lax.axis_index("c")
    sid = lax.axis_index("s")
    wid = core * NS + sid
    pltpu.sync_copy(bias_hbm, bias_v)
    r0a = wid * RB
    r0b = (wid + NW) * RB
    act1 = (wid + NW) < NRC

    def issue_in(r0, V0, V1, D0, D1):
        pltpu.async_copy(vals0_hbm.at[pl.ds(r0, RB), :], V0, insem)
        pltpu.async_copy(vals1_hbm.at[pl.ds(r0, RB), :], V1, insem)
        pltpu.async_copy(den0_hbm.at[pl.ds(r0, RB)], D0.at[pl.ds(0, RB)], insem)
        pltpu.async_copy(den1_hbm.at[pl.ds(r0, RB)], D1.at[pl.ds(0, RB)], insem)

    def wait_in(r0, V0, V1, D0, D1):
        pltpu.make_async_copy(vals0_hbm.at[pl.ds(r0, RB), :], V0, insem).wait()
        pltpu.make_async_copy(vals1_hbm.at[pl.ds(r0, RB), :], V1, insem).wait()
        pltpu.make_async_copy(den0_hbm.at[pl.ds(r0, RB)], D0.at[pl.ds(0, RB)],
                              insem).wait()
        pltpu.make_async_copy(den1_hbm.at[pl.ds(r0, RB)], D1.at[pl.ds(0, RB)],
                              insem).wait()

    def compute(V0, V1, D0, D1):
        for g in range(208 // L):
            sl = pl.ds(g * L, L)
            dv = D0[sl] + D1[sl]
            recip_v[sl] = jnp.where(dv != 0.0, 1.0 / dv, 0.0)

        def _row(r, _):
            rv = recip_v[pl.ds(r, L)]
            wv = jnp.full((L,), rv[0], jnp.float32)
            for j in range(H // L):
                sl = pl.ds(j * L, L)
                o = (V0[r, sl] + V1[r, sl]) * wv + bias_v[sl]
                o = jnp.where(o > 0.0, o, jnp.exp(o) - 1.0)
                V0[r, sl] = o
            return 0
        lax.fori_loop(0, RB, _row, 0)

    issue_in(r0a, v0a, v1a, d0a, d1a)

    @pl.when(act1)
    def _():
        issue_in(r0b, v0b, v1b, d0b, d1b)

    wait_in(r0a, v0a, v1a, d0a, d1a)
    compute(v0a, v1a, d0a, d1a)
    pltpu.async_copy(v0a, out_hbm.at[0, pl.ds(r0a, RB), :], outsem)

    @pl.when(act1)
    def _():
        wait_in(r0b, v0b, v1b, d0b, d1b)
        compute(v0b, v1b, d0b, d1b)
        pltpu.async_copy(v0b, out_hbm.at[0, pl.ds(r0b, RB), :], outsem)
        pltpu.make_async_copy(v0b, out_hbm.at[0, pl.ds(r0b, RB), :],
                              outsem).wait()

    pltpu.make_async_copy(v0a, out_hbm.at[0, pl.ds(r0a, RB), :], outsem).wait()


_sc2 = functools.partial(
    pl.kernel, _sc2_body,
    compiler_params=_SC_PARAMS,
    out_type=jax.ShapeDtypeStruct((1, N, H), jnp.float32),
    mesh=_MESH,
    scratch_types=(
        pltpu.VMEM((RB, H), jnp.float32),
        pltpu.VMEM((RB, H), jnp.float32),
        pltpu.VMEM((RB, H), jnp.float32),
        pltpu.VMEM((RB, H), jnp.float32),
        pltpu.VMEM((208,), jnp.float32),
        pltpu.VMEM((208,), jnp.float32),
        pltpu.VMEM((208,), jnp.float32),
        pltpu.VMEM((208,), jnp.float32),
        pltpu.VMEM((224,), jnp.float32),
        pltpu.VMEM((H,), jnp.float32),
        pltpu.SemaphoreType.DMA,
        pltpu.SemaphoreType.DMA,
    ),
)()


def kernel(seq, edge_index, training, W, a1, b1, a2, b2, bias_zero):
    x = seq[0]
    rows = edge_index[0]
    cols = edge_index[1]
    A = jnp.concatenate([a1, a2], axis=1)
    b2d = jnp.concatenate([b1, b2]).reshape(1, 2)
    sfts, f12, m12 = _tc1(x, W, A, b2d)
    f1 = f12[:, 0]
    f2 = f12[:, 1]
    mpad = jnp.pad(m12.reshape(2), (0, L - 2))
    vals0, vals1, den0, den1 = _sc1b(rows, cols, f1, f2, mpad, sfts)
    return _sc2(vals0, vals1, den0, den1, bias_zero)


# speedup vs baseline: 43.1577x; 1.0478x over previous
import functools

import jax
import jax.numpy as jnp
from jax import lax
from jax.experimental import pallas as pl
from jax.experimental.pallas import tpu as pltpu
from jax.experimental.pallas import tpu_sc as plsc

N = 10000
E = 320000
D = 128
H = 128

NC = 2
NS = 16
NW = NC * NS
L = 16

NP = 10240
EPT = E // NW
C = 80
NCHUNK = EPT // C
CA = 400
NCA = EPT // CA

TCB = 2000


def _tc1_body(x_ref, w_ref, a_ref, b_ref, sfts_ref, f12_ref, m_ref):
    i = pl.program_id(0)
    s = jnp.dot(x_ref[...], w_ref[...],
                precision=lax.Precision.HIGHEST,
                preferred_element_type=jnp.float32)
    sfts_ref[...] = s
    f = jnp.dot(s, a_ref[...],
                precision=lax.Precision.HIGHEST,
                preferred_element_type=jnp.float32) + b_ref[...]
    f12_ref[...] = f
    m = jnp.max(f, axis=0, keepdims=True)

    @pl.when(i == 0)
    def _():
        m_ref[...] = m

    @pl.when(i != 0)
    def _():
        m_ref[...] = jnp.maximum(m_ref[...], m)


def _tc1(x, W, A, b2d):
    return pl.pallas_call(
        _tc1_body,
        grid=(N // TCB,),
        in_specs=[
            pl.BlockSpec((TCB, D), lambda i: (i, 0)),
            pl.BlockSpec((D, H), lambda i: (0, 0)),
            pl.BlockSpec((H, 2), lambda i: (0, 0)),
            pl.BlockSpec((1, 2), lambda i: (0, 0)),
        ],
        out_specs=[
            pl.BlockSpec((TCB, H), lambda i: (i, 0)),
            pl.BlockSpec((TCB, 2), lambda i: (i, 0)),
            pl.BlockSpec((1, 2), lambda i: (0, 0)),
        ],
        out_shape=[
            jax.ShapeDtypeStruct((N, H), jnp.float32),
            jax.ShapeDtypeStruct((N, 2), jnp.float32),
            jax.ShapeDtypeStruct((1, 2), jnp.float32),
        ],
    )(x, W, A, b2d)


_MESH = plsc.VectorSubcoreMesh(
    core_axis_name="c", subcore_axis_name="s", num_cores=NC, num_subcores=NS)

_SC_PARAMS = pltpu.CompilerParams(
    needs_layout_passes=False, use_tc_tiling_on_sc=False)


def _sc1b_body(rows_hbm, cols_hbm, f1_hbm, f2_hbm, m_hbm, sfts_hbm,
               vals0_out, vals1_out, den0_out, den1_out,
               rv0, rv1, rv2, cv0, cv1, cv2,
               fa0, fa1, fa2, fb0, fb1, fb2, m_v,
               rs0, rs1, rs2, g0, g1, g2, x0, x1, x2, dent,
               vals_sh, den16_sh, gsem, isem, fsem, ssem0, ssem1, ssem2):
    core = lax.axis_index("c")
    sid = lax.axis_index("s")
    tbase = (core * NS + sid) * EPT
    RV = [rv0, rv1, rv2]
    CV = [cv0, cv1, cv2]
    FA = [fa0, fa1, fa2]
    FB = [fb0, fb1, fb2]
    RS = [rs0, rs1, rs2]
    G = [g0, g1, g2]
    X16 = [x0, x1, x2]
    SS = [ssem0, ssem1, ssem2]
    zeros16f = jnp.zeros((L,), jnp.float32)
    zeros16i = jnp.zeros((L,), jnp.int32)
    iota16 = lax.iota(jnp.int32, L)

    def _z1(r, _):
        for j in range(D // L):
            g0[r, pl.ds(j * L, L)] = zeros16f
        x0[r, :] = zeros16f
        return 0
    lax.fori_loop(0, C, _z1, 0)
    rows_per_tile = NP // NS
    for k in range(rows_per_tile // C):
        pltpu.sync_copy(g0, vals_sh.at[pl.ds(sid * rows_per_tile + k * C, C), :])
        pltpu.sync_copy(x0, den16_sh.at[pl.ds(sid * rows_per_tile + k * C, C), :])
    plsc.subcore_barrier()

    def issue_idx(k, P):
        sl = pl.ds(tbase + k * C, C)
        pltpu.async_copy(rows_hbm.at[sl], RV[P], isem)
        pltpu.async_copy(cols_hbm.at[sl], CV[P], isem)

    def wait_idx(k, P):
        sl = pl.ds(tbase + k * C, C)
        pltpu.make_async_copy(rows_hbm.at[sl], RV[P], isem).wait()
        pltpu.make_async_copy(cols_hbm.at[sl], CV[P], isem).wait()

    def issue_fg(P):
        pltpu.async_copy(f1_hbm.at[RV[P]], FA[P], fsem)
        pltpu.async_copy(f2_hbm.at[CV[P]], FB[P], fsem)

    def wait_fg(P):
        pltpu.make_async_copy(f1_hbm.at[RV[P]], FA[P], fsem).wait()
        pltpu.make_async_copy(f2_hbm.at[CV[P]], FB[P], fsem).wait()

    def wait_scatter(R):
        pltpu.make_async_copy(G[R], vals_sh.at[RS[R]], SS[R]).wait()
        pltpu.make_async_copy(X16[R], den16_sh.at[RS[R]], SS[R]).wait()

    pltpu.sync_copy(m_hbm, m_v)
    mrow = m_v[:]
    c_shift = jnp.maximum(mrow[0] + mrow[1], 0.0)

    issue_idx(0, 0)
    issue_idx(1, 1)
    wait_idx(0, 0)
    issue_fg(0)
    pltpu.async_copy(sfts_hbm.at[CV[0]], G[0], gsem)

    def _body(k, P, Q, R):
        pltpu.make_async_copy(sfts_hbm.at[CV[P]], G[P], gsem).wait()
        wait_fg(P)

        @pl.when(k + 1 < NCHUNK)
        def _():
            wait_idx(k + 1, Q)
            issue_fg(Q)

        @pl.when(k + 2 < NCHUNK)
        def _():
            issue_idx(k + 2, R)

        @pl.when(k >= 1)
        def _():
            wait_scatter(R)

        @pl.when(k + 1 < NCHUNK)
        def _():
            pltpu.async_copy(sfts_hbm.at[CV[Q]], G[Q], gsem)

        for g in range(C // L):
            sl = pl.ds(g * L, L)
            RS[P][sl] = RV[P][sl]

        def _grp(g, _):
            sl16 = pl.ds(g * L, L)
            e = FA[P][sl16] + FB[P][sl16]
            e = jnp.where(e >= 0.0, e, 0.2 * e) - c_shift
            exvec = jnp.exp(e)
            plsc.store_scatter(X16[P], [iota16 + g * L, zeros16i], exvec)
            for lane in range(L):
                wv = jnp.full((L,), exvec[lane], jnp.float32)
                ei = g * L + lane
                for j in range(D // L):
                    sl = pl.ds(j * L, L)
                    G[P][ei, sl] = G[P][ei, sl] * wv
            return 0
        lax.fori_loop(0, C // L, _grp, 0)

        pltpu.async_copy(G[P], vals_sh.at[RS[P]], SS[P], add=True)
        pltpu.async_copy(X16[P], den16_sh.at[RS[P]], SS[P], add=True)

    def _full(k, _):
        for ph in range(3):
            @pl.when(k % 3 == ph)
            def _():
                _body(k, ph, (ph + 1) % 3, (ph + 2) % 3)
        return 0
    lax.fori_loop(0, NCHUNK, _full, 0)

    wait_scatter((NCHUNK - 1) % 3)
    plsc.subcore_barrier()

    drows = NP // NS
    for kk in range(drows // C):
        pltpu.sync_copy(den16_sh.at[pl.ds(sid * drows + kk * C, C), :], x0)
        for g in range(C // L):
            dv = plsc.load_gather(x0, [iota16 + g * L, zeros16i])
            for lane in range(L):
                ei = g * L + lane
                wv = jnp.full((L,), dv[lane], jnp.float32)
                for j in range(D // L):
                    g0[ei, pl.ds(j * L, L)] = wv

        @pl.when(core == 0)
        def _():
            pltpu.sync_copy(
                g0, den0_out.at[pl.ds(sid * drows + kk * C, C), :])

        @pl.when(core == 1)
        def _():
            pltpu.sync_copy(
                g0, den1_out.at[pl.ds(sid * drows + kk * C, C), :])

    @pl.when(core == 0)
    def _():
        pltpu.sync_copy(vals_sh.at[pl.ds(sid * drows, drows), :],
                        vals0_out.at[pl.ds(sid * drows, drows), :])

    @pl.when(core == 1)
    def _():
        pltpu.sync_copy(vals_sh.at[pl.ds(sid * drows, drows), :],
                        vals1_out.at[pl.ds(sid * drows, drows), :])


_sc1b = functools.partial(
    pl.kernel, _sc1b_body,
    compiler_params=_SC_PARAMS,
    out_type=(
        jax.ShapeDtypeStruct((NP, H), jnp.float32),
        jax.ShapeDtypeStruct((NP, H), jnp.float32),
        jax.ShapeDtypeStruct((NP, H), jnp.float32),
        jax.ShapeDtypeStruct((NP, H), jnp.float32),
    ),
    mesh=_MESH,
    scratch_types=(
        pltpu.VMEM((C,), jnp.int32),
        pltpu.VMEM((C,), jnp.int32),
        pltpu.VMEM((C,), jnp.int32),
        pltpu.VMEM((C,), jnp.int32),
        pltpu.VMEM((C,), jnp.int32),
        pltpu.VMEM((C,), jnp.int32),
        pltpu.VMEM((C,), jnp.float32),
        pltpu.VMEM((C,), jnp.float32),
        pltpu.VMEM((C,), jnp.float32),
        pltpu.VMEM((C,), jnp.float32),
        pltpu.VMEM((C,), jnp.float32),
        pltpu.VMEM((C,), jnp.float32),
        pltpu.VMEM((L,), jnp.float32),
        pltpu.VMEM((C,), jnp.int32),
        pltpu.VMEM((C,), jnp.int32),
        pltpu.VMEM((C,), jnp.int32),
        pltpu.VMEM((C, D), jnp.float32),
        pltpu.VMEM((C, D), jnp.float32),
        pltpu.VMEM((C, D), jnp.float32),
        pltpu.VMEM((C, L), jnp.float32),
        pltpu.VMEM((C, L), jnp.float32),
        pltpu.VMEM((C, L), jnp.float32),
        pltpu.VMEM((NP // NS,), jnp.float32),
        pltpu.VMEM_SHARED((NP, H), jnp.float32),
        pltpu.VMEM_SHARED((NP, L), jnp.float32),
        pltpu.SemaphoreType.DMA,
        pltpu.SemaphoreType.DMA,
        pltpu.SemaphoreType.DMA,
        pltpu.SemaphoreType.DMA,
        pltpu.SemaphoreType.DMA,
        pltpu.SemaphoreType.DMA,
    ),
)()


def _tc2_body(v0_ref, v1_ref, de0_ref, de1_ref, bias_ref, out_ref):
    v = v0_ref[...] + v1_ref[...]
    d = de0_ref[...] + de1_ref[...]
    o = v * jnp.where(d != 0.0, 1.0 / d, 0.0) + bias_ref[...]
    out_ref[0] = jnp.where(o > 0.0, o, jnp.exp(jnp.minimum(o, 0.0)) - 1.0)


def _tc2(vals0, vals1, den0, den1, bias2d):
    return pl.pallas_call(
        _tc2_body,
        grid=(N // TCB,),
        in_specs=[
            pl.BlockSpec((TCB, H), lambda i: (i, 0)),
            pl.BlockSpec((TCB, H), lambda i: (i, 0)),
            pl.BlockSpec((TCB, H), lambda i: (i, 0)),
            pl.BlockSpec((TCB, H), lambda i: (i, 0)),
            pl.BlockSpec((1, H), lambda i: (0, 0)),
        ],
        out_specs=pl.BlockSpec((1, TCB, H), lambda i: (0, i, 0)),
        out_shape=jax.ShapeDtypeStruct((1, N, H), jnp.float32),
    )(vals0, vals1, den0, den1, bias2d)


def kernel(seq, edge_index, training, W, a1, b1, a2, b2, bias_zero):
    x = seq[0]
    rows = edge_index[0]
    cols = edge_index[1]
    A = jnp.concatenate([a1, a2], axis=1)
    b2d = jnp.concatenate([b1, b2]).reshape(1, 2)
    sfts, f12, m12 = _tc1(x, W, A, b2d)
    f1 = f12[:, 0]
    f2 = f12[:, 1]
    mpad = jnp.pad(m12.reshape(2), (0, L - 2))
    vals0, vals1, den0, den1 = _sc1b(rows, cols, f1, f2, mpad, sfts)
    return _tc2(vals0[:N], vals1[:N], den0[:N], den1[:N],
                bias_zero.reshape(1, H))
